# baseline (speedup 1.0000x reference)
import functools

import jax
import jax.numpy as jnp
from jax import lax
from jax.experimental import pallas as pl
from jax.experimental.pallas import tpu as pltpu
from jax.experimental.pallas import tpu_sc as plsc

N = 10000
D_IN = 256
H1, C1 = 8, 16
F1 = H1 * C1
AUG1 = F1 + 16
AUG2 = 16
NP = 10016
NC, NS = 2, 16
NW = NC * NS
CH1 = 64
CH2 = 64
BR = 2504
Q0_FRAC = 0.357


def _split(etot):
    qsum = -(-etot // (NS * CH1))
    qsum = -(-qsum // 4) * 4
    q0 = int(round(qsum * Q0_FRAC / 4.0)) * 4
    q0 = min(max(q0, 4), qsum - 4)
    return q0, qsum - q0


def _prep1_body(x_ref, w1_ref, asm_ref, adm_ref, haug_ref, adst_ref):
    h = jnp.dot(x_ref[...], w1_ref[...], preferred_element_type=jnp.float32)
    asrc = jnp.dot(h, asm_ref[...], preferred_element_type=jnp.float32)
    adst = jnp.dot(h, adm_ref[...], preferred_element_type=jnp.float32)
    z8 = jnp.zeros_like(asrc)
    haug_ref[...] = jnp.concatenate([h, asrc, z8], axis=1)
    adst_ref[...] = jnp.concatenate([adst, z8], axis=1)


def _prep1(xp, W1, AS, AD):
    grid = NP // BR
    return pl.pallas_call(
        _prep1_body,
        grid=(grid,),
        in_specs=[
            pl.BlockSpec((BR, D_IN), lambda i: (i, 0)),
            pl.BlockSpec((D_IN, F1), lambda i: (0, 0)),
            pl.BlockSpec((F1, H1), lambda i: (0, 0)),
            pl.BlockSpec((F1, H1), lambda i: (0, 0)),
        ],
        out_specs=[
            pl.BlockSpec((BR, AUG1), lambda i: (i, 0)),
            pl.BlockSpec((BR, 16), lambda i: (i, 0)),
        ],
        out_shape=[
            jax.ShapeDtypeStruct((NP, AUG1), jnp.float32),
            jax.ShapeDtypeStruct((NP, 16), jnp.float32),
        ],
    )(xp, W1, AS, AD)


def _edge_sc(haug, adstt, srcc, dstc, augw, ch, q0, q1, make_edge):
    mesh = plsc.VectorSubcoreMesh(core_axis_name="c", subcore_axis_name="s",
                                  num_cores=NC, num_subcores=NS)
    rpt = NP // NS
    assert q0 % 4 == 0 and q1 % 4 == 0 and min(q0, q1) >= 8

    @functools.partial(
        pl.kernel,
        out_type=jax.ShapeDtypeStruct((NC * NP, augw), jnp.float32),
        mesh=mesh,
        scratch_types=[
            pltpu.VMEM((4, ch), jnp.int32),
            pltpu.VMEM((4, ch), jnp.int32),
            pltpu.VMEM((2, ch, augw), jnp.float32),
            pltpu.VMEM((2, ch, 16), jnp.float32),
            pltpu.VMEM((ch, augw), jnp.float32),
            pltpu.VMEM_SHARED((NP, augw), jnp.float32),
        ] + [pltpu.SemaphoreType.DMA] * 13,
        compiler_params=pltpu.CompilerParams(use_tc_tiling_on_sc=False),
    )
    def k(haug_hbm, adst_hbm, src_hbm, dst_hbm, out_hbm,
          sib, dib, rows, rowd, msg, acc_sh, *sems_all):
        c = lax.axis_index("c")
        s = lax.axis_index("s")
        semis = list(sems_all[0:4])
        semid = list(sems_all[4:8])
        semg = list(sems_all[8:10])
        semd = list(sems_all[10:12])
        sems = sems_all[12]
        qc = jnp.where(c == 0, q0, q1)
        base_chunk = c * (NS * q0) + s * qc

        zero16 = jnp.zeros((16,), jnp.float32)

        def zrow(r, carry):
            for kk in range(augw // 16):
                msg[r, pl.ds(kk * 16, 16)] = zero16
            return carry
        lax.fori_loop(0, ch, zrow, 0)
        off0 = 0
        for sz in ([ch] * (rpt // ch) + ([rpt % ch] if rpt % ch else [])):
            pltpu.sync_copy(msg.at[pl.ds(0, sz)],
                            acc_sh.at[pl.ds(s * rpt + off0, sz)])
            off0 += sz
        plsc.subcore_barrier()

        def fetch_idx(chunk, d):
            pltpu.async_copy(src_hbm.at[base_chunk + chunk], sib.at[d],
                             semis[d])
            pltpu.async_copy(dst_hbm.at[base_chunk + chunk], dib.at[d],
                             semid[d])

        def wait_idx(d):
            pltpu.make_async_copy(src_hbm.at[0], sib.at[d], semis[d]).wait()
            pltpu.make_async_copy(dst_hbm.at[0], dib.at[d], semid[d]).wait()

        def fetch_rows(d, gb):
            pltpu.async_copy(haug_hbm.at[sib.at[d]], rows.at[gb], semg[gb])
            pltpu.async_copy(adst_hbm.at[dib.at[d]], rowd.at[gb], semd[gb])

        def wait_rows(gb):
            pltpu.make_async_copy(haug_hbm.at[pl.ds(0, ch)], rows.at[gb],
                                  semg[gb]).wait()
            pltpu.make_async_copy(adst_hbm.at[pl.ds(0, ch)], rowd.at[gb],
                                  semd[gb]).wait()

        def wait_scatter():
            pltpu.make_async_copy(haug_hbm.at[pl.ds(0, ch)], msg, sems).wait()

        for d in range(4):
            fetch_idx(d, d)
        for d in range(2):
            wait_idx(d)
            fetch_rows(d, d)

        def handle(g4, b):
            geff = g4 + b
            gb = b & 1
            wait_rows(gb)

            @pl.when(geff >= 1)
            def _():
                wait_scatter()

            @pl.when((geff >= 1) & (geff + 3 < qc))
            def _():
                fetch_idx(geff + 3, (b + 3) & 3)

            edge = make_edge(rows, rowd, msg, gb)
            plsc.parallel_loop(0, ch, unroll=4)(edge)

            pltpu.async_copy(msg, acc_sh.at[dib.at[b]], sems, add=True)

            @pl.when(geff + 2 < qc)
            def _():
                wait_idx((b + 2) & 3)
                fetch_rows((b + 2) & 3, gb)

        def quad(p, carry):
            g4 = p * 4
            for b in range(4):
                handle(g4, b)
            return carry
        lax.fori_loop(0, qc // 4, quad, 0)

        wait_scatter()
        plsc.subcore_barrier()
        off1 = 0
        for sz in ([ch] * (rpt // ch) + ([rpt % ch] if rpt % ch else [])):
            off = s * rpt + off1
            pltpu.sync_copy(acc_sh.at[pl.ds(off, sz)],
                            out_hbm.at[pl.ds(c * NP + off, sz)])
            off1 += sz

    return k(haug, adstt, srcc, dstc)


def _make_edge1(rows, rowd, msg, b):
    iota = lax.iota(jnp.int32, 16)
    mask8 = jnp.where(iota < H1, 1.0, 0.0).astype(jnp.float32)

    def edge(i):
        asrc = rows[b, i, pl.ds(F1, 16)]
        adst = rowd[b, i, pl.ds(0, 16)]
        e = asrc + adst
        e = jnp.where(e >= 0.0, e, e * 0.2)
        w = jnp.exp(e)
        for kk in range(H1):
            msg[i, pl.ds(kk * 16, 16)] = w[kk] * rows[b, i, pl.ds(kk * 16, 16)]
        msg[i, pl.ds(F1, 16)] = w * mask8
    return edge


def _make_edge2(rows, rowd, msg, b):
    iota = lax.iota(jnp.int32, 16)

    def edge(i):
        rs = rows[b, i, pl.ds(0, 16)]
        rd = rowd[b, i, pl.ds(0, 16)]
        ev = (rs[2] + rd[0]) + jnp.zeros((16,), jnp.float32)
        ev = jnp.where(ev >= 0.0, ev, ev * 0.2)
        w = jnp.exp(ev)
        sel = jnp.where(iota == 2, 1.0, rs)
        msg[i, pl.ds(0, 16)] = w * sel
    return edge


def _mid_body(p0_ref, p1_ref, exp8_ref, b1_ref, w2_ref, a2s_ref, a2d_ref,
              haug2_ref, adst2_ref):
    ssum = p0_ref[...] + p1_ref[...]
    num = ssum[:, :F1]
    den = ssum[:, F1:F1 + H1]
    rec = 1.0 / (den + 1e-16)
    rec128 = jnp.dot(rec, exp8_ref[...], preferred_element_type=jnp.float32)
    o1 = num * rec128 + b1_ref[...]
    g = jnp.where(o1 > 0.0, o1, jnp.exp(o1) - 1.0)
    h2 = jnp.dot(g, w2_ref[...], preferred_element_type=jnp.float32)
    s2 = jnp.dot(h2, a2s_ref[...], preferred_element_type=jnp.float32)
    d2 = jnp.dot(h2, a2d_ref[...], preferred_element_type=jnp.float32)
    zb = jnp.zeros((h2.shape[0], 13), jnp.float32)
    zc = jnp.zeros((h2.shape[0], 15), jnp.float32)
    haug2_ref[...] = jnp.concatenate([h2, s2, zb], axis=1)
    adst2_ref[...] = jnp.concatenate([d2, zc], axis=1)


def _mid(part1, EXP8, b1r, W2, a2s, a2d):
    grid = NP // BR
    return pl.pallas_call(
        _mid_body,
        grid=(grid,),
        in_specs=[
            pl.BlockSpec((BR, AUG1), lambda i: (i, 0)),
            pl.BlockSpec((BR, AUG1), lambda i: (i + NP // BR, 0)),
            pl.BlockSpec((H1, F1), lambda i: (0, 0)),
            pl.BlockSpec((1, F1), lambda i: (0, 0)),
            pl.BlockSpec((F1, 2), lambda i: (0, 0)),
            pl.BlockSpec((2, 1), lambda i: (0, 0)),
            pl.BlockSpec((2, 1), lambda i: (0, 0)),
        ],
        out_specs=[
            pl.BlockSpec((BR, AUG2), lambda i: (i, 0)),
            pl.BlockSpec((BR, AUG2), lambda i: (i, 0)),
        ],
        out_shape=[
            jax.ShapeDtypeStruct((NP, AUG2), jnp.float32),
            jax.ShapeDtypeStruct((NP, AUG2), jnp.float32),
        ],
    )(part1, part1, EXP8, b1r, W2, a2s, a2d)


def _fin_body(p0_ref, p1_ref, b2_ref, out_ref):
    ssum = p0_ref[...] + p1_ref[...]
    out_ref[...] = ssum[:, 0:2] / (ssum[:, 2:3] + 1e-16) + b2_ref[...]


def _fin(part2, b2r):
    grid = NP // BR
    return pl.pallas_call(
        _fin_body,
        grid=(grid,),
        in_specs=[
            pl.BlockSpec((BR, AUG2), lambda i: (i, 0)),
            pl.BlockSpec((BR, AUG2), lambda i: (i + NP // BR, 0)),
            pl.BlockSpec((1, 2), lambda i: (0, 0)),
        ],
        out_specs=pl.BlockSpec((BR, 2), lambda i: (i, 0)),
        out_shape=jax.ShapeDtypeStruct((NP, 2), jnp.float32),
    )(part2, part2, b2r)


def kernel(x, edge_index, W1, a1_src, a1_dst, b1, W2, a2_src, a2_dst, b2):
    E = edge_index.shape[1]
    loops = jnp.arange(N, dtype=jnp.int32)
    etot = E + N
    q0, q1 = _split(etot)
    qsum = q0 + q1
    qmax = max(q0, q1)
    ep = NS * qsum * CH1
    epad = ep + qmax * CH1
    junk = jnp.full((epad - etot,), N, jnp.int32)
    srcf = jnp.concatenate([edge_index[0].astype(jnp.int32), loops, junk])
    dstf = jnp.concatenate([edge_index[1].astype(jnp.int32), loops, junk])
    src64 = srcf.reshape(-1, CH1)
    dst64 = dstf.reshape(-1, CH1)
    src128 = srcf.reshape(-1, CH2)
    dst128 = dstf.reshape(-1, CH2)
    assert CH2 == CH1

    xp = jnp.pad(x, ((0, NP - N), (0, 0)))
    eye = jnp.eye(H1, dtype=jnp.float32)
    AS = (a1_src[0][:, :, None] * eye[:, None, :]).reshape(F1, H1)
    AD = (a1_dst[0][:, :, None] * eye[:, None, :]).reshape(F1, H1)

    haug1, adst1 = _prep1(xp, W1, AS, AD)
    part1 = _edge_sc(haug1, adst1, src64, dst64, AUG1, CH1, q0, q1,
                     _make_edge1)

    EXP8 = (jnp.arange(F1)[None, :] // C1 == jnp.arange(H1)[:, None]
            ).astype(jnp.float32)
    haug2, adst2 = _mid(part1, EXP8, b1.reshape(1, F1), W2,
                        a2_src.reshape(2, 1), a2_dst.reshape(2, 1))
    part2 = _edge_sc(haug2, adst2, src128, dst128, AUG2, CH2,
                     q0, q1, _make_edge2)
    outp = _fin(part2, b2.reshape(1, 2))
    return outp[:N]


# speedup vs baseline: 1.0765x; 1.0765x over previous
import functools

import jax
import jax.numpy as jnp
from jax import lax
from jax.experimental import pallas as pl
from jax.experimental.pallas import tpu as pltpu
from jax.experimental.pallas import tpu_sc as plsc

N = 10000
D_IN = 256
H1, C1 = 8, 16
F1 = H1 * C1
AUG1 = F1 + 16
AUG2 = 16
NP = 10016
NC, NS = 2, 16
NW = NC * NS
CH1 = 64
CH2 = 64
BR = 2504
Q0_FRAC = 0.50


def _split(etot):
    qsum = -(-etot // (NS * CH1))
    qsum = -(-qsum // 4) * 4
    q0 = int(round(qsum * Q0_FRAC / 4.0)) * 4
    q0 = min(max(q0, 4), qsum - 4)
    return q0, qsum - q0


def _prep1_body(x_ref, w1_ref, asm_ref, adm_ref, haug_ref, adst_ref):
    h = jnp.dot(x_ref[...], w1_ref[...], preferred_element_type=jnp.float32)
    asrc = jnp.dot(h, asm_ref[...], preferred_element_type=jnp.float32)
    adst = jnp.dot(h, adm_ref[...], preferred_element_type=jnp.float32)
    z8 = jnp.zeros_like(asrc)
    haug_ref[...] = jnp.concatenate([h, asrc, z8], axis=1)
    adst_ref[...] = jnp.concatenate([adst, z8], axis=1)


def _prep1(xp, W1, AS, AD):
    grid = NP // BR
    return pl.pallas_call(
        _prep1_body,
        grid=(grid,),
        in_specs=[
            pl.BlockSpec((BR, D_IN), lambda i: (i, 0)),
            pl.BlockSpec((D_IN, F1), lambda i: (0, 0)),
            pl.BlockSpec((F1, H1), lambda i: (0, 0)),
            pl.BlockSpec((F1, H1), lambda i: (0, 0)),
        ],
        out_specs=[
            pl.BlockSpec((BR, AUG1), lambda i: (i, 0)),
            pl.BlockSpec((BR, 16), lambda i: (i, 0)),
        ],
        out_shape=[
            jax.ShapeDtypeStruct((NP, AUG1), jnp.float32),
            jax.ShapeDtypeStruct((NP, 16), jnp.float32),
        ],
    )(xp, W1, AS, AD)


def _edge_sc(haug, adstt, srcc, dstc, augw, ch, q0, q1, make_edge):
    mesh = plsc.VectorSubcoreMesh(core_axis_name="c", subcore_axis_name="s",
                                  num_cores=NC, num_subcores=NS)
    rpt = NP // NS
    assert q0 % 4 == 0 and q1 % 4 == 0 and min(q0, q1) >= 8

    @functools.partial(
        pl.kernel,
        out_type=jax.ShapeDtypeStruct((NC * NP, augw), jnp.float32),
        mesh=mesh,
        scratch_types=[
            pltpu.VMEM((4, ch), jnp.int32),
            pltpu.VMEM((4, ch), jnp.int32),
            pltpu.VMEM((2, ch, augw), jnp.float32),
            pltpu.VMEM((2, ch, 16), jnp.float32),
            pltpu.VMEM((ch, augw), jnp.float32),
            pltpu.VMEM_SHARED((NP, augw), jnp.float32),
        ] + [pltpu.SemaphoreType.DMA] * 13,
        compiler_params=pltpu.CompilerParams(use_tc_tiling_on_sc=False),
    )
    def k(haug_hbm, adst_hbm, src_hbm, dst_hbm, out_hbm,
          sib, dib, rows, rowd, msg, acc_sh, *sems_all):
        c = lax.axis_index("c")
        s = lax.axis_index("s")
        semis = list(sems_all[0:4])
        semid = list(sems_all[4:8])
        semg = list(sems_all[8:10])
        semd = list(sems_all[10:12])
        sems = sems_all[12]
        qc = jnp.where(c == 0, q0, q1)
        base_chunk = c * (NS * q0) + s * qc

        zero16 = jnp.zeros((16,), jnp.float32)

        def zrow(r, carry):
            for kk in range(augw // 16):
                msg[r, pl.ds(kk * 16, 16)] = zero16
            return carry
        lax.fori_loop(0, ch, zrow, 0)
        off0 = 0
        for sz in ([ch] * (rpt // ch) + ([rpt % ch] if rpt % ch else [])):
            pltpu.sync_copy(msg.at[pl.ds(0, sz)],
                            acc_sh.at[pl.ds(s * rpt + off0, sz)])
            off0 += sz
        plsc.subcore_barrier()

        def fetch_idx(chunk, d):
            pltpu.async_copy(src_hbm.at[base_chunk + chunk], sib.at[d],
                             semis[d])
            pltpu.async_copy(dst_hbm.at[base_chunk + chunk], dib.at[d],
                             semid[d])

        def wait_idx(d):
            pltpu.make_async_copy(src_hbm.at[0], sib.at[d], semis[d]).wait()
            pltpu.make_async_copy(dst_hbm.at[0], dib.at[d], semid[d]).wait()

        def fetch_rows(d, gb):
            pltpu.async_copy(haug_hbm.at[sib.at[d]], rows.at[gb], semg[gb])
            pltpu.async_copy(adst_hbm.at[dib.at[d]], rowd.at[gb], semd[gb])

        def wait_rows(gb):
            pltpu.make_async_copy(haug_hbm.at[pl.ds(0, ch)], rows.at[gb],
                                  semg[gb]).wait()
            pltpu.make_async_copy(adst_hbm.at[pl.ds(0, ch)], rowd.at[gb],
                                  semd[gb]).wait()

        def wait_scatter():
            pltpu.make_async_copy(haug_hbm.at[pl.ds(0, ch)], msg, sems).wait()

        for d in range(4):
            fetch_idx(d, d)
        for d in range(2):
            wait_idx(d)
            fetch_rows(d, d)

        def handle(g4, b):
            geff = g4 + b
            gb = b & 1
            wait_rows(gb)

            @pl.when(geff >= 1)
            def _():
                wait_scatter()

            @pl.when((geff >= 1) & (geff + 3 < qc))
            def _():
                fetch_idx(geff + 3, (b + 3) & 3)

            edge = make_edge(rows, rowd, msg, gb)
            plsc.parallel_loop(0, ch, unroll=4)(edge)

            pltpu.async_copy(msg, acc_sh.at[dib.at[b]], sems, add=True)

            @pl.when(geff + 2 < qc)
            def _():
                wait_idx((b + 2) & 3)
                fetch_rows((b + 2) & 3, gb)

        def quad(p, carry):
            g4 = p * 4
            for b in range(4):
                handle(g4, b)
            return carry
        lax.fori_loop(0, qc // 4, quad, 0)

        wait_scatter()
        plsc.subcore_barrier()
        off1 = 0
        for sz in ([ch] * (rpt // ch) + ([rpt % ch] if rpt % ch else [])):
            off = s * rpt + off1
            pltpu.sync_copy(acc_sh.at[pl.ds(off, sz)],
                            out_hbm.at[pl.ds(c * NP + off, sz)])
            off1 += sz

    return k(haug, adstt, srcc, dstc)


def _make_edge1(rows, rowd, msg, b):
    iota = lax.iota(jnp.int32, 16)
    mask8 = jnp.where(iota < H1, 1.0, 0.0).astype(jnp.float32)

    def edge(i):
        asrc = rows[b, i, pl.ds(F1, 16)]
        adst = rowd[b, i, pl.ds(0, 16)]
        e = asrc + adst
        e = jnp.where(e >= 0.0, e, e * 0.2)
        w = jnp.exp(e)
        for kk in range(H1):
            msg[i, pl.ds(kk * 16, 16)] = w[kk] * rows[b, i, pl.ds(kk * 16, 16)]
        msg[i, pl.ds(F1, 16)] = w * mask8
    return edge


def _make_edge2(rows, rowd, msg, b):
    iota = lax.iota(jnp.int32, 16)

    def edge(i):
        rs = rows[b, i, pl.ds(0, 16)]
        rd = rowd[b, i, pl.ds(0, 16)]
        ev = (rs[2] + rd[0]) + jnp.zeros((16,), jnp.float32)
        ev = jnp.where(ev >= 0.0, ev, ev * 0.2)
        w = jnp.exp(ev)
        sel = jnp.where(iota == 2, 1.0, rs)
        msg[i, pl.ds(0, 16)] = w * sel
    return edge


def _mid_body(p0_ref, p1_ref, exp8_ref, b1_ref, w2_ref, a2s_ref, a2d_ref,
              haug2_ref, adst2_ref):
    ssum = p0_ref[...] + p1_ref[...]
    num = ssum[:, :F1]
    den = ssum[:, F1:F1 + H1]
    rec = 1.0 / (den + 1e-16)
    rec128 = jnp.dot(rec, exp8_ref[...], preferred_element_type=jnp.float32)
    o1 = num * rec128 + b1_ref[...]
    g = jnp.where(o1 > 0.0, o1, jnp.exp(o1) - 1.0)
    h2 = jnp.dot(g, w2_ref[...], preferred_element_type=jnp.float32)
    s2 = jnp.dot(h2, a2s_ref[...], preferred_element_type=jnp.float32)
    d2 = jnp.dot(h2, a2d_ref[...], preferred_element_type=jnp.float32)
    zb = jnp.zeros((h2.shape[0], 13), jnp.float32)
    zc = jnp.zeros((h2.shape[0], 15), jnp.float32)
    haug2_ref[...] = jnp.concatenate([h2, s2, zb], axis=1)
    adst2_ref[...] = jnp.concatenate([d2, zc], axis=1)


def _mid(part1, EXP8, b1r, W2, a2s, a2d):
    grid = NP // BR
    return pl.pallas_call(
        _mid_body,
        grid=(grid,),
        in_specs=[
            pl.BlockSpec((BR, AUG1), lambda i: (i, 0)),
            pl.BlockSpec((BR, AUG1), lambda i: (i + NP // BR, 0)),
            pl.BlockSpec((H1, F1), lambda i: (0, 0)),
            pl.BlockSpec((1, F1), lambda i: (0, 0)),
            pl.BlockSpec((F1, 2), lambda i: (0, 0)),
            pl.BlockSpec((2, 1), lambda i: (0, 0)),
            pl.BlockSpec((2, 1), lambda i: (0, 0)),
        ],
        out_specs=[
            pl.BlockSpec((BR, AUG2), lambda i: (i, 0)),
            pl.BlockSpec((BR, AUG2), lambda i: (i, 0)),
        ],
        out_shape=[
            jax.ShapeDtypeStruct((NP, AUG2), jnp.float32),
            jax.ShapeDtypeStruct((NP, AUG2), jnp.float32),
        ],
    )(part1, part1, EXP8, b1r, W2, a2s, a2d)


def _fin_body(p0_ref, p1_ref, b2_ref, out_ref):
    ssum = p0_ref[...] + p1_ref[...]
    out_ref[...] = ssum[:, 0:2] / (ssum[:, 2:3] + 1e-16) + b2_ref[...]


def _fin(part2, b2r):
    grid = NP // BR
    return pl.pallas_call(
        _fin_body,
        grid=(grid,),
        in_specs=[
            pl.BlockSpec((BR, AUG2), lambda i: (i, 0)),
            pl.BlockSpec((BR, AUG2), lambda i: (i + NP // BR, 0)),
            pl.BlockSpec((1, 2), lambda i: (0, 0)),
        ],
        out_specs=pl.BlockSpec((BR, 2), lambda i: (i, 0)),
        out_shape=jax.ShapeDtypeStruct((NP, 2), jnp.float32),
    )(part2, part2, b2r)


def kernel(x, edge_index, W1, a1_src, a1_dst, b1, W2, a2_src, a2_dst, b2):
    E = edge_index.shape[1]
    loops = jnp.arange(N, dtype=jnp.int32)
    etot = E + N
    q0, q1 = _split(etot)
    qsum = q0 + q1
    qmax = max(q0, q1)
    ep = NS * qsum * CH1
    epad = ep + qmax * CH1
    junk = jnp.full((epad - etot,), N, jnp.int32)
    srcf = jnp.concatenate([edge_index[0].astype(jnp.int32), loops, junk])
    dstf = jnp.concatenate([edge_index[1].astype(jnp.int32), loops, junk])
    src64 = srcf.reshape(-1, CH1)
    dst64 = dstf.reshape(-1, CH1)
    src128 = srcf.reshape(-1, CH2)
    dst128 = dstf.reshape(-1, CH2)
    assert CH2 == CH1

    xp = jnp.pad(x, ((0, NP - N), (0, 0)))
    eye = jnp.eye(H1, dtype=jnp.float32)
    AS = (a1_src[0][:, :, None] * eye[:, None, :]).reshape(F1, H1)
    AD = (a1_dst[0][:, :, None] * eye[:, None, :]).reshape(F1, H1)

    haug1, adst1 = _prep1(xp, W1, AS, AD)
    part1 = _edge_sc(haug1, adst1, src64, dst64, AUG1, CH1, q0, q1,
                     _make_edge1)

    EXP8 = (jnp.arange(F1)[None, :] // C1 == jnp.arange(H1)[:, None]
            ).astype(jnp.float32)
    haug2, adst2 = _mid(part1, EXP8, b1.reshape(1, F1), W2,
                        a2_src.reshape(2, 1), a2_dst.reshape(2, 1))
    part2 = _edge_sc(haug2, adst2, src128, dst128, AUG2, CH2,
                     q0, q1, _make_edge2)
    outp = _fin(part2, b2.reshape(1, 2))
    return outp[:N]


# speedup vs baseline: 1.1696x; 1.0865x over previous
import functools

import jax
import jax.numpy as jnp
from jax import lax
from jax.experimental import pallas as pl
from jax.experimental.pallas import tpu as pltpu
from jax.experimental.pallas import tpu_sc as plsc

N = 10000
D_IN = 256
H1, C1 = 8, 16
F1 = H1 * C1
AUG1 = F1 + 16
AUG2 = 16
NP = 10016
NC, NS = 2, 16
NW = NC * NS
CH = 64
BR = 2504
Q0_FRAC = 0.643
RING = 8


def _split(etot):
    qsum = -(-etot // (NS * CH))
    qsum = -(-qsum // RING) * RING
    q0 = int(round(qsum * Q0_FRAC / RING)) * RING
    q0 = min(max(q0, RING), qsum - RING)
    return q0, qsum - q0


def _prep1_body(x_ref, w1_ref, asm_ref, adm_ref, haug_ref, adst_ref):
    h = jnp.dot(x_ref[...], w1_ref[...], preferred_element_type=jnp.float32)
    asrc = jnp.dot(h, asm_ref[...], preferred_element_type=jnp.float32)
    adst = jnp.dot(h, adm_ref[...], preferred_element_type=jnp.float32)
    z8 = jnp.zeros_like(asrc)
    haug_ref[...] = jnp.concatenate([h, asrc, z8], axis=1)
    adst_ref[...] = jnp.concatenate([adst, z8], axis=1)


def _prep1(xp, W1, AS, AD):
    grid = NP // BR
    return pl.pallas_call(
        _prep1_body,
        grid=(grid,),
        in_specs=[
            pl.BlockSpec((BR, D_IN), lambda i: (i, 0)),
            pl.BlockSpec((D_IN, F1), lambda i: (0, 0)),
            pl.BlockSpec((F1, H1), lambda i: (0, 0)),
            pl.BlockSpec((F1, H1), lambda i: (0, 0)),
        ],
        out_specs=[
            pl.BlockSpec((BR, AUG1), lambda i: (i, 0)),
            pl.BlockSpec((BR, 16), lambda i: (i, 0)),
        ],
        out_shape=[
            jax.ShapeDtypeStruct((NP, AUG1), jnp.float32),
            jax.ShapeDtypeStruct((NP, 16), jnp.float32),
        ],
    )(xp, W1, AS, AD)


def _edge_sc(haug, adstt, srcc, dstc, augw, ch, q0, q1, make_edge):
    mesh = plsc.VectorSubcoreMesh(core_axis_name="c", subcore_axis_name="s",
                                  num_cores=NC, num_subcores=NS)
    rpt = NP // NS
    assert q0 % RING == 0 and q1 % RING == 0 and min(q0, q1) >= RING

    @functools.partial(
        pl.kernel,
        out_type=jax.ShapeDtypeStruct((NC * NP, augw), jnp.float32),
        mesh=mesh,
        scratch_types=[
            pltpu.VMEM((RING, ch), jnp.int32),
            pltpu.VMEM((RING, ch), jnp.int32),
            pltpu.VMEM((2, ch, augw), jnp.float32),
            pltpu.VMEM((2, ch, 16), jnp.float32),
            pltpu.VMEM((ch, augw), jnp.float32),
            pltpu.VMEM_SHARED((NP, augw), jnp.float32),
        ] + [pltpu.SemaphoreType.DMA] * (2 * RING + 5),
        compiler_params=pltpu.CompilerParams(use_tc_tiling_on_sc=False),
    )
    def k(haug_hbm, adst_hbm, src_hbm, dst_hbm, out_hbm,
          sib, dib, rows, rowd, msg, acc_sh, *sems_all):
        c = lax.axis_index("c")
        s = lax.axis_index("s")
        semis = list(sems_all[0:RING])
        semid = list(sems_all[RING:2 * RING])
        semg = list(sems_all[2 * RING:2 * RING + 2])
        semd = list(sems_all[2 * RING + 2:2 * RING + 4])
        sems = sems_all[2 * RING + 4]
        qc = jnp.where(c == 0, q0, q1)
        base_chunk = c * (NS * q0) + s * qc

        zero16 = jnp.zeros((16,), jnp.float32)

        def zrow(r, carry):
            for kk in range(augw // 16):
                msg[r, pl.ds(kk * 16, 16)] = zero16
            return carry
        lax.fori_loop(0, ch, zrow, 0)
        off0 = 0
        for sz in ([ch] * (rpt // ch) + ([rpt % ch] if rpt % ch else [])):
            pltpu.sync_copy(msg.at[pl.ds(0, sz)],
                            acc_sh.at[pl.ds(s * rpt + off0, sz)])
            off0 += sz
        plsc.subcore_barrier()

        def fetch_idx(chunk, d):
            pltpu.async_copy(src_hbm.at[base_chunk + chunk], sib.at[d],
                             semis[d])
            pltpu.async_copy(dst_hbm.at[base_chunk + chunk], dib.at[d],
                             semid[d])

        def wait_idx(d):
            pltpu.make_async_copy(src_hbm.at[0], sib.at[d], semis[d]).wait()
            pltpu.make_async_copy(dst_hbm.at[0], dib.at[d], semid[d]).wait()

        def fetch_rows(d, gb):
            pltpu.async_copy(haug_hbm.at[sib.at[d]], rows.at[gb], semg[gb])
            pltpu.async_copy(adst_hbm.at[dib.at[d]], rowd.at[gb], semd[gb])

        def wait_rows(gb):
            pltpu.make_async_copy(haug_hbm.at[pl.ds(0, ch)], rows.at[gb],
                                  semg[gb]).wait()
            pltpu.make_async_copy(adst_hbm.at[pl.ds(0, ch)], rowd.at[gb],
                                  semd[gb]).wait()

        def wait_scatter():
            pltpu.make_async_copy(haug_hbm.at[pl.ds(0, ch)], msg, sems).wait()

        for d in range(RING):
            fetch_idx(d, d)
        for d in range(2):
            wait_idx(d)
            fetch_rows(d, d)

        def handle(g8, b):
            geff = g8 + b
            gb = b & 1
            wait_rows(gb)

            @pl.when(geff >= 1)
            def _():
                wait_scatter()

            @pl.when((geff >= 1) & (geff + (RING - 1) < qc))
            def _():
                fetch_idx(geff + (RING - 1), (b + RING - 1) % RING)

            edge = make_edge(rows, rowd, msg, gb)
            plsc.parallel_loop(0, ch, unroll=4)(edge)

            pltpu.async_copy(msg, acc_sh.at[dib.at[b]], sems, add=True)

            @pl.when(geff + 2 < qc)
            def _():
                wait_idx((b + 2) % RING)
                fetch_rows((b + 2) % RING, gb)

        def octet(p, carry):
            g8 = p * RING
            for b in range(RING):
                handle(g8, b)
            return carry
        lax.fori_loop(0, qc // RING, octet, 0)

        wait_scatter()
        plsc.subcore_barrier()
        off1 = 0
        for sz in ([ch] * (rpt // ch) + ([rpt % ch] if rpt % ch else [])):
            off = s * rpt + off1
            pltpu.sync_copy(acc_sh.at[pl.ds(off, sz)],
                            out_hbm.at[pl.ds(c * NP + off, sz)])
            off1 += sz

    return k(haug, adstt, srcc, dstc)


def _make_edge1(rows, rowd, msg, b):
    iota = lax.iota(jnp.int32, 16)
    mask8 = jnp.where(iota < H1, 1.0, 0.0).astype(jnp.float32)

    def edge(i):
        asrc = rows[b, i, pl.ds(F1, 16)]
        adst = rowd[b, i, pl.ds(0, 16)]
        e = asrc + adst
        e = jnp.where(e >= 0.0, e, e * 0.2)
        w = jnp.exp(e)
        for kk in range(H1):
            msg[i, pl.ds(kk * 16, 16)] = w[kk] * rows[b, i, pl.ds(kk * 16, 16)]
        msg[i, pl.ds(F1, 16)] = w * mask8
    return edge


def _make_edge2(rows, rowd, msg, b):
    iota = lax.iota(jnp.int32, 16)

    def edge(i):
        rs = rows[b, i, pl.ds(0, 16)]
        rd = rowd[b, i, pl.ds(0, 16)]
        ev = (rs[2] + rd[0]) + jnp.zeros((16,), jnp.float32)
        ev = jnp.where(ev >= 0.0, ev, ev * 0.2)
        w = jnp.exp(ev)
        sel = jnp.where(iota == 2, 1.0, rs)
        msg[i, pl.ds(0, 16)] = w * sel
    return edge


def _mid_body(p0_ref, p1_ref, exp8_ref, b1_ref, w2_ref, a2s_ref, a2d_ref,
              haug2_ref, adst2_ref):
    ssum = p0_ref[...] + p1_ref[...]
    num = ssum[:, :F1]
    den = ssum[:, F1:F1 + H1]
    rec = 1.0 / (den + 1e-16)
    rec128 = jnp.dot(rec, exp8_ref[...], preferred_element_type=jnp.float32)
    o1 = num * rec128 + b1_ref[...]
    g = jnp.where(o1 > 0.0, o1, jnp.exp(o1) - 1.0)
    h2 = jnp.dot(g, w2_ref[...], preferred_element_type=jnp.float32)
    s2 = jnp.dot(h2, a2s_ref[...], preferred_element_type=jnp.float32)
    d2 = jnp.dot(h2, a2d_ref[...], preferred_element_type=jnp.float32)
    zb = jnp.zeros((h2.shape[0], 13), jnp.float32)
    zc = jnp.zeros((h2.shape[0], 15), jnp.float32)
    haug2_ref[...] = jnp.concatenate([h2, s2, zb], axis=1)
    adst2_ref[...] = jnp.concatenate([d2, zc], axis=1)


def _mid(part1, EXP8, b1r, W2, a2s, a2d):
    grid = NP // BR
    return pl.pallas_call(
        _mid_body,
        grid=(grid,),
        in_specs=[
            pl.BlockSpec((BR, AUG1), lambda i: (i, 0)),
            pl.BlockSpec((BR, AUG1), lambda i: (i + NP // BR, 0)),
            pl.BlockSpec((H1, F1), lambda i: (0, 0)),
            pl.BlockSpec((1, F1), lambda i: (0, 0)),
            pl.BlockSpec((F1, 2), lambda i: (0, 0)),
            pl.BlockSpec((2, 1), lambda i: (0, 0)),
            pl.BlockSpec((2, 1), lambda i: (0, 0)),
        ],
        out_specs=[
            pl.BlockSpec((BR, AUG2), lambda i: (i, 0)),
            pl.BlockSpec((BR, AUG2), lambda i: (i, 0)),
        ],
        out_shape=[
            jax.ShapeDtypeStruct((NP, AUG2), jnp.float32),
            jax.ShapeDtypeStruct((NP, AUG2), jnp.float32),
        ],
    )(part1, part1, EXP8, b1r, W2, a2s, a2d)


def _fin_body(p0_ref, p1_ref, b2_ref, out_ref):
    ssum = p0_ref[...] + p1_ref[...]
    out_ref[...] = ssum[:, 0:2] / (ssum[:, 2:3] + 1e-16) + b2_ref[...]


def _fin(part2, b2r):
    grid = NP // BR
    return pl.pallas_call(
        _fin_body,
        grid=(grid,),
        in_specs=[
            pl.BlockSpec((BR, AUG2), lambda i: (i, 0)),
            pl.BlockSpec((BR, AUG2), lambda i: (i + NP // BR, 0)),
            pl.BlockSpec((1, 2), lambda i: (0, 0)),
        ],
        out_specs=pl.BlockSpec((BR, 2), lambda i: (i, 0)),
        out_shape=jax.ShapeDtypeStruct((NP, 2), jnp.float32),
    )(part2, part2, b2r)


def kernel(x, edge_index, W1, a1_src, a1_dst, b1, W2, a2_src, a2_dst, b2):
    E = edge_index.shape[1]
    loops = jnp.arange(N, dtype=jnp.int32)
    etot = E + N
    q0, q1 = _split(etot)
    qsum = q0 + q1
    ep = NS * qsum * CH
    junk = jnp.full((ep - etot,), N, jnp.int32)
    srcf = jnp.concatenate([edge_index[0].astype(jnp.int32), loops, junk])
    dstf = jnp.concatenate([edge_index[1].astype(jnp.int32), loops, junk])
    srcc = srcf.reshape(-1, CH)
    dstc = dstf.reshape(-1, CH)

    xp = jnp.pad(x, ((0, NP - N), (0, 0)))
    eye = jnp.eye(H1, dtype=jnp.float32)
    AS = (a1_src[0][:, :, None] * eye[:, None, :]).reshape(F1, H1)
    AD = (a1_dst[0][:, :, None] * eye[:, None, :]).reshape(F1, H1)

    haug1, adst1 = _prep1(xp, W1, AS, AD)
    part1 = _edge_sc(haug1, adst1, srcc, dstc, AUG1, CH, q0, q1, _make_edge1)

    EXP8 = (jnp.arange(F1)[None, :] // C1 == jnp.arange(H1)[:, None]
            ).astype(jnp.float32)
    haug2, adst2 = _mid(part1, EXP8, b1.reshape(1, F1), W2,
                        a2_src.reshape(2, 1), a2_dst.reshape(2, 1))
    part2 = _edge_sc(haug2, adst2, srcc, dstc, AUG2, CH, q0, q1, _make_edge2)
    outp = _fin(part2, b2.reshape(1, 2))
    return outp[:N]


# speedup vs baseline: 1.2050x; 1.0302x over previous
import functools

import jax
import jax.numpy as jnp
from jax import lax
from jax.experimental import pallas as pl
from jax.experimental.pallas import tpu as pltpu
from jax.experimental.pallas import tpu_sc as plsc

N = 10000
D_IN = 256
H1, C1 = 8, 16
F1 = H1 * C1
AUG1 = F1 + 16
AUG2 = 16
NP = 10016
NC, NS = 2, 16
NW = NC * NS
CH = 64
BR = 2504
Q0_FRAC1 = 0.667
Q0_FRAC2 = 0.50
RING = 8


def _split(etot, frac):
    qsum = -(-etot // (NS * CH))
    qsum = -(-qsum // RING) * RING
    q0 = int(round(qsum * frac / RING)) * RING
    q0 = min(max(q0, RING), qsum - RING)
    return q0, qsum - q0


def _prep1_body(x_ref, w1_ref, asm_ref, adm_ref, haug_ref, adst_ref):
    h = jnp.dot(x_ref[...], w1_ref[...], preferred_element_type=jnp.float32)
    asrc = jnp.dot(h, asm_ref[...], preferred_element_type=jnp.float32)
    adst = jnp.dot(h, adm_ref[...], preferred_element_type=jnp.float32)
    z8 = jnp.zeros_like(asrc)
    haug_ref[...] = jnp.concatenate([h, asrc, z8], axis=1)
    adst_ref[...] = jnp.concatenate([adst, z8], axis=1)


def _prep1(xp, W1, AS, AD):
    grid = NP // BR
    return pl.pallas_call(
        _prep1_body,
        grid=(grid,),
        in_specs=[
            pl.BlockSpec((BR, D_IN), lambda i: (i, 0)),
            pl.BlockSpec((D_IN, F1), lambda i: (0, 0)),
            pl.BlockSpec((F1, H1), lambda i: (0, 0)),
            pl.BlockSpec((F1, H1), lambda i: (0, 0)),
        ],
        out_specs=[
            pl.BlockSpec((BR, AUG1), lambda i: (i, 0)),
            pl.BlockSpec((BR, 16), lambda i: (i, 0)),
        ],
        out_shape=[
            jax.ShapeDtypeStruct((NP, AUG1), jnp.float32),
            jax.ShapeDtypeStruct((NP, 16), jnp.float32),
        ],
    )(xp, W1, AS, AD)


def _edge_sc(haug, adstt, srcc, dstc, augw, ch, q0, q1, make_edge):
    mesh = plsc.VectorSubcoreMesh(core_axis_name="c", subcore_axis_name="s",
                                  num_cores=NC, num_subcores=NS)
    rpt = NP // NS
    assert q0 % RING == 0 and q1 % RING == 0 and min(q0, q1) >= RING

    @functools.partial(
        pl.kernel,
        out_type=jax.ShapeDtypeStruct((NC * NP, augw), jnp.float32),
        mesh=mesh,
        scratch_types=[
            pltpu.VMEM((RING, ch), jnp.int32),
            pltpu.VMEM((RING, ch), jnp.int32),
            pltpu.VMEM((2, ch, augw), jnp.float32),
            pltpu.VMEM((2, ch, 16), jnp.float32),
            pltpu.VMEM((ch, augw), jnp.float32),
            pltpu.VMEM_SHARED((NP, augw), jnp.float32),
        ] + [pltpu.SemaphoreType.DMA] * (2 * RING + 5),
        compiler_params=pltpu.CompilerParams(use_tc_tiling_on_sc=False),
    )
    def k(haug_hbm, adst_hbm, src_hbm, dst_hbm, out_hbm,
          sib, dib, rows, rowd, msg, acc_sh, *sems_all):
        c = lax.axis_index("c")
        s = lax.axis_index("s")
        semis = list(sems_all[0:RING])
        semid = list(sems_all[RING:2 * RING])
        semg = list(sems_all[2 * RING:2 * RING + 2])
        semd = list(sems_all[2 * RING + 2:2 * RING + 4])
        sems = sems_all[2 * RING + 4]
        qc = jnp.where(c == 0, q0, q1)
        base_chunk = c * (NS * q0) + s * qc

        zero16 = jnp.zeros((16,), jnp.float32)

        def zrow(r, carry):
            for kk in range(augw // 16):
                msg[r, pl.ds(kk * 16, 16)] = zero16
            return carry
        lax.fori_loop(0, ch, zrow, 0)
        off0 = 0
        for sz in ([ch] * (rpt // ch) + ([rpt % ch] if rpt % ch else [])):
            pltpu.sync_copy(msg.at[pl.ds(0, sz)],
                            acc_sh.at[pl.ds(s * rpt + off0, sz)])
            off0 += sz
        plsc.subcore_barrier()

        def fetch_idx(chunk, d):
            pltpu.async_copy(src_hbm.at[base_chunk + chunk], sib.at[d],
                             semis[d])
            pltpu.async_copy(dst_hbm.at[base_chunk + chunk], dib.at[d],
                             semid[d])

        def wait_idx(d):
            pltpu.make_async_copy(src_hbm.at[0], sib.at[d], semis[d]).wait()
            pltpu.make_async_copy(dst_hbm.at[0], dib.at[d], semid[d]).wait()

        def fetch_rows(d, gb):
            pltpu.async_copy(haug_hbm.at[sib.at[d]], rows.at[gb], semg[gb])
            pltpu.async_copy(adst_hbm.at[dib.at[d]], rowd.at[gb], semd[gb])

        def wait_rows(gb):
            pltpu.make_async_copy(haug_hbm.at[pl.ds(0, ch)], rows.at[gb],
                                  semg[gb]).wait()
            pltpu.make_async_copy(adst_hbm.at[pl.ds(0, ch)], rowd.at[gb],
                                  semd[gb]).wait()

        def wait_scatter():
            pltpu.make_async_copy(haug_hbm.at[pl.ds(0, ch)], msg, sems).wait()

        for d in range(RING):
            fetch_idx(d, d)
        for d in range(2):
            wait_idx(d)
            fetch_rows(d, d)

        def handle(g8, b):
            geff = g8 + b
            gb = b & 1
            wait_rows(gb)

            @pl.when(geff >= 1)
            def _():
                wait_scatter()

            @pl.when((geff >= 1) & (geff + (RING - 1) < qc))
            def _():
                fetch_idx(geff + (RING - 1), (b + RING - 1) % RING)

            edge = make_edge(rows, rowd, msg, gb)
            plsc.parallel_loop(0, ch, unroll=4)(edge)

            pltpu.async_copy(msg, acc_sh.at[dib.at[b]], sems, add=True)

            @pl.when(geff + 2 < qc)
            def _():
                wait_idx((b + 2) % RING)
                fetch_rows((b + 2) % RING, gb)

        def octet(p, carry):
            g8 = p * RING
            for b in range(RING):
                handle(g8, b)
            return carry
        lax.fori_loop(0, qc // RING, octet, 0)

        wait_scatter()
        plsc.subcore_barrier()
        off1 = 0
        for sz in ([ch] * (rpt // ch) + ([rpt % ch] if rpt % ch else [])):
            off = s * rpt + off1
            pltpu.sync_copy(acc_sh.at[pl.ds(off, sz)],
                            out_hbm.at[pl.ds(c * NP + off, sz)])
            off1 += sz

    return k(haug, adstt, srcc, dstc)


def _make_edge1(rows, rowd, msg, b):
    iota = lax.iota(jnp.int32, 16)
    mask8 = jnp.where(iota < H1, 1.0, 0.0).astype(jnp.float32)

    def edge(i):
        asrc = rows[b, i, pl.ds(F1, 16)]
        adst = rowd[b, i, pl.ds(0, 16)]
        e = asrc + adst
        e = jnp.where(e >= 0.0, e, e * 0.2)
        w = jnp.exp(e)
        for kk in range(H1):
            msg[i, pl.ds(kk * 16, 16)] = w[kk] * rows[b, i, pl.ds(kk * 16, 16)]
        msg[i, pl.ds(F1, 16)] = w * mask8
    return edge


def _make_edge2(rows, rowd, msg, b):
    iota = lax.iota(jnp.int32, 16)

    def edge(i):
        rs = rows[b, i, pl.ds(0, 16)]
        rd = rowd[b, i, pl.ds(0, 16)]
        ev = (rs[2] + rd[0]) + jnp.zeros((16,), jnp.float32)
        ev = jnp.where(ev >= 0.0, ev, ev * 0.2)
        w = jnp.exp(ev)
        sel = jnp.where(iota == 2, 1.0, rs)
        msg[i, pl.ds(0, 16)] = w * sel
    return edge


def _mid_body(p0_ref, p1_ref, exp8_ref, b1_ref, w2_ref, a2s_ref, a2d_ref,
              haug2_ref, adst2_ref):
    ssum = p0_ref[...] + p1_ref[...]
    num = ssum[:, :F1]
    den = ssum[:, F1:F1 + H1]
    rec = 1.0 / (den + 1e-16)
    rec128 = jnp.dot(rec, exp8_ref[...], preferred_element_type=jnp.float32)
    o1 = num * rec128 + b1_ref[...]
    g = jnp.where(o1 > 0.0, o1, jnp.exp(o1) - 1.0)
    h2 = jnp.dot(g, w2_ref[...], preferred_element_type=jnp.float32)
    s2 = jnp.dot(h2, a2s_ref[...], preferred_element_type=jnp.float32)
    d2 = jnp.dot(h2, a2d_ref[...], preferred_element_type=jnp.float32)
    zb = jnp.zeros((h2.shape[0], 13), jnp.float32)
    zc = jnp.zeros((h2.shape[0], 15), jnp.float32)
    haug2_ref[...] = jnp.concatenate([h2, s2, zb], axis=1)
    adst2_ref[...] = jnp.concatenate([d2, zc], axis=1)


def _mid(part1, EXP8, b1r, W2, a2s, a2d):
    grid = NP // BR
    return pl.pallas_call(
        _mid_body,
        grid=(grid,),
        in_specs=[
            pl.BlockSpec((BR, AUG1), lambda i: (i, 0)),
            pl.BlockSpec((BR, AUG1), lambda i: (i + NP // BR, 0)),
            pl.BlockSpec((H1, F1), lambda i: (0, 0)),
            pl.BlockSpec((1, F1), lambda i: (0, 0)),
            pl.BlockSpec((F1, 2), lambda i: (0, 0)),
            pl.BlockSpec((2, 1), lambda i: (0, 0)),
            pl.BlockSpec((2, 1), lambda i: (0, 0)),
        ],
        out_specs=[
            pl.BlockSpec((BR, AUG2), lambda i: (i, 0)),
            pl.BlockSpec((BR, AUG2), lambda i: (i, 0)),
        ],
        out_shape=[
            jax.ShapeDtypeStruct((NP, AUG2), jnp.float32),
            jax.ShapeDtypeStruct((NP, AUG2), jnp.float32),
        ],
    )(part1, part1, EXP8, b1r, W2, a2s, a2d)


def _fin_body(p0_ref, p1_ref, b2_ref, out_ref):
    ssum = p0_ref[...] + p1_ref[...]
    out_ref[...] = ssum[:, 0:2] / (ssum[:, 2:3] + 1e-16) + b2_ref[...]


def _fin(part2, b2r):
    grid = NP // BR
    return pl.pallas_call(
        _fin_body,
        grid=(grid,),
        in_specs=[
            pl.BlockSpec((BR, AUG2), lambda i: (i, 0)),
            pl.BlockSpec((BR, AUG2), lambda i: (i + NP // BR, 0)),
            pl.BlockSpec((1, 2), lambda i: (0, 0)),
        ],
        out_specs=pl.BlockSpec((BR, 2), lambda i: (i, 0)),
        out_shape=jax.ShapeDtypeStruct((NP, 2), jnp.float32),
    )(part2, part2, b2r)


def kernel(x, edge_index, W1, a1_src, a1_dst, b1, W2, a2_src, a2_dst, b2):
    E = edge_index.shape[1]
    loops = jnp.arange(N, dtype=jnp.int32)
    etot = E + N
    q0, q1 = _split(etot, Q0_FRAC1)
    q2a, q2b = _split(etot, Q0_FRAC2)
    qsum = q0 + q1
    ep = NS * qsum * CH
    junk = jnp.full((ep - etot,), N, jnp.int32)
    srcf = jnp.concatenate([edge_index[0].astype(jnp.int32), loops, junk])
    dstf = jnp.concatenate([edge_index[1].astype(jnp.int32), loops, junk])
    srcc = srcf.reshape(-1, CH)
    dstc = dstf.reshape(-1, CH)

    xp = jnp.pad(x, ((0, NP - N), (0, 0)))
    eye = jnp.eye(H1, dtype=jnp.float32)
    AS = (a1_src[0][:, :, None] * eye[:, None, :]).reshape(F1, H1)
    AD = (a1_dst[0][:, :, None] * eye[:, None, :]).reshape(F1, H1)

    haug1, adst1 = _prep1(xp, W1, AS, AD)
    part1 = _edge_sc(haug1, adst1, srcc, dstc, AUG1, CH, q0, q1, _make_edge1)

    EXP8 = (jnp.arange(F1)[None, :] // C1 == jnp.arange(H1)[:, None]
            ).astype(jnp.float32)
    haug2, adst2 = _mid(part1, EXP8, b1.reshape(1, F1), W2,
                        a2_src.reshape(2, 1), a2_dst.reshape(2, 1))
    part2 = _edge_sc(haug2, adst2, srcc, dstc, AUG2, CH, q2a, q2b, _make_edge2)
    outp = _fin(part2, b2.reshape(1, 2))
    return outp[:N]


# speedup vs baseline: 1.2303x; 1.0210x over previous
import functools

import jax
import jax.numpy as jnp
from jax import lax
from jax.experimental import pallas as pl
from jax.experimental.pallas import tpu as pltpu
from jax.experimental.pallas import tpu_sc as plsc

N = 10000
D_IN = 256
H1, C1 = 8, 16
F1 = H1 * C1
AUG1 = F1 + 16
AUG2 = 16
NP = 10016
NC, NS = 2, 16
NW = NC * NS
CH = 64
BR = 2504
Q0_FRAC1 = 0.714
Q0_FRAC2 = 0.50
RING = 8


def _split(etot, frac):
    qsum = -(-etot // (NS * CH))
    qsum = -(-qsum // RING) * RING
    q0 = int(round(qsum * frac / RING)) * RING
    q0 = min(max(q0, RING), qsum - RING)
    return q0, qsum - q0


def _prep1_body(x_ref, w1_ref, asm_ref, adm_ref, haug_ref, adst_ref):
    h = jnp.dot(x_ref[...], w1_ref[...], preferred_element_type=jnp.float32)
    asrc = jnp.dot(h, asm_ref[...], preferred_element_type=jnp.float32)
    adst = jnp.dot(h, adm_ref[...], preferred_element_type=jnp.float32)
    z8 = jnp.zeros_like(asrc)
    haug_ref[...] = jnp.concatenate([h, asrc, z8], axis=1)
    adst_ref[...] = jnp.concatenate([adst, z8], axis=1)


def _prep1(xp, W1, AS, AD):
    grid = NP // BR
    return pl.pallas_call(
        _prep1_body,
        grid=(grid,),
        in_specs=[
            pl.BlockSpec((BR, D_IN), lambda i: (i, 0)),
            pl.BlockSpec((D_IN, F1), lambda i: (0, 0)),
            pl.BlockSpec((F1, H1), lambda i: (0, 0)),
            pl.BlockSpec((F1, H1), lambda i: (0, 0)),
        ],
        out_specs=[
            pl.BlockSpec((BR, AUG1), lambda i: (i, 0)),
            pl.BlockSpec((BR, 16), lambda i: (i, 0)),
        ],
        out_shape=[
            jax.ShapeDtypeStruct((NP, AUG1), jnp.float32),
            jax.ShapeDtypeStruct((NP, 16), jnp.float32),
        ],
    )(xp, W1, AS, AD)


def _edge_sc(haug, adstt, srcc, dstc, augw, ch, q0, q1, make_edge):
    mesh = plsc.VectorSubcoreMesh(core_axis_name="c", subcore_axis_name="s",
                                  num_cores=NC, num_subcores=NS)
    rpt = NP // NS
    assert q0 % RING == 0 and q1 % RING == 0 and min(q0, q1) >= RING

    @functools.partial(
        pl.kernel,
        out_type=jax.ShapeDtypeStruct((NC * NP, augw), jnp.float32),
        mesh=mesh,
        scratch_types=[
            pltpu.VMEM((RING, ch), jnp.int32),
            pltpu.VMEM((RING, ch), jnp.int32),
            pltpu.VMEM((2, ch, augw), jnp.float32),
            pltpu.VMEM((2, ch, 16), jnp.float32),
            pltpu.VMEM((ch, augw), jnp.float32),
            pltpu.VMEM_SHARED((NP, augw), jnp.float32),
        ] + [pltpu.SemaphoreType.DMA] * (2 * RING + 5),
        compiler_params=pltpu.CompilerParams(use_tc_tiling_on_sc=False),
    )
    def k(haug_hbm, adst_hbm, src_hbm, dst_hbm, out_hbm,
          sib, dib, rows, rowd, msg, acc_sh, *sems_all):
        c = lax.axis_index("c")
        s = lax.axis_index("s")
        semis = list(sems_all[0:RING])
        semid = list(sems_all[RING:2 * RING])
        semg = list(sems_all[2 * RING:2 * RING + 2])
        semd = list(sems_all[2 * RING + 2:2 * RING + 4])
        sems = sems_all[2 * RING + 4]
        qc = jnp.where(c == 0, q0, q1)
        base_chunk = c * (NS * q0) + s * qc

        zero16 = jnp.zeros((16,), jnp.float32)

        def zrow(r, carry):
            for kk in range(augw // 16):
                msg[r, pl.ds(kk * 16, 16)] = zero16
            return carry
        lax.fori_loop(0, ch, zrow, 0)
        off0 = 0
        for sz in ([ch] * (rpt // ch) + ([rpt % ch] if rpt % ch else [])):
            pltpu.sync_copy(msg.at[pl.ds(0, sz)],
                            acc_sh.at[pl.ds(s * rpt + off0, sz)])
            off0 += sz
        plsc.subcore_barrier()

        def fetch_idx(chunk, d):
            pltpu.async_copy(src_hbm.at[base_chunk + chunk], sib.at[d],
                             semis[d])
            pltpu.async_copy(dst_hbm.at[base_chunk + chunk], dib.at[d],
                             semid[d])

        def wait_idx(d):
            pltpu.make_async_copy(src_hbm.at[0], sib.at[d], semis[d]).wait()
            pltpu.make_async_copy(dst_hbm.at[0], dib.at[d], semid[d]).wait()

        def fetch_rows(d, gb):
            pltpu.async_copy(haug_hbm.at[sib.at[d]], rows.at[gb], semg[gb])
            pltpu.async_copy(adst_hbm.at[dib.at[d]], rowd.at[gb], semd[gb])

        def wait_rows(gb):
            pltpu.make_async_copy(haug_hbm.at[pl.ds(0, ch)], rows.at[gb],
                                  semg[gb]).wait()
            pltpu.make_async_copy(adst_hbm.at[pl.ds(0, ch)], rowd.at[gb],
                                  semd[gb]).wait()

        def wait_scatter():
            pltpu.make_async_copy(haug_hbm.at[pl.ds(0, ch)], msg, sems).wait()

        for d in range(RING):
            fetch_idx(d, d)
        for d in range(2):
            wait_idx(d)
            fetch_rows(d, d)

        def handle(g8, b):
            geff = g8 + b
            gb = b & 1
            wait_rows(gb)

            @pl.when(geff >= 1)
            def _():
                wait_scatter()

            @pl.when((geff >= 1) & (geff + (RING - 1) < qc))
            def _():
                fetch_idx(geff + (RING - 1), (b + RING - 1) % RING)

            edge = make_edge(rows, rowd, msg, gb)
            plsc.parallel_loop(0, ch, unroll=4)(edge)

            pltpu.async_copy(msg, acc_sh.at[dib.at[b]], sems, add=True)

            @pl.when(geff + 2 < qc)
            def _():
                wait_idx((b + 2) % RING)
                fetch_rows((b + 2) % RING, gb)

        def octet(p, carry):
            g8 = p * RING
            for b in range(RING):
                handle(g8, b)
            return carry
        lax.fori_loop(0, qc // RING, octet, 0)

        wait_scatter()
        plsc.subcore_barrier()
        off1 = 0
        for sz in ([ch] * (rpt // ch) + ([rpt % ch] if rpt % ch else [])):
            off = s * rpt + off1
            pltpu.sync_copy(acc_sh.at[pl.ds(off, sz)],
                            out_hbm.at[pl.ds(c * NP + off, sz)])
            off1 += sz

    return k(haug, adstt, srcc, dstc)


def _make_edge1(rows, rowd, msg, b):
    iota = lax.iota(jnp.int32, 16)
    mask8 = jnp.where(iota < H1, 1.0, 0.0).astype(jnp.float32)

    def edge(i):
        asrc = rows[b, i, pl.ds(F1, 16)]
        adst = rowd[b, i, pl.ds(0, 16)]
        e = asrc + adst
        e = jnp.where(e >= 0.0, e, e * 0.2)
        w = jnp.exp(e)
        for kk in range(H1):
            msg[i, pl.ds(kk * 16, 16)] = w[kk] * rows[b, i, pl.ds(kk * 16, 16)]
        msg[i, pl.ds(F1, 16)] = w * mask8
    return edge


def _make_edge2(rows, rowd, msg, b):
    iota = lax.iota(jnp.int32, 16)

    def edge(i):
        rs = rows[b, i, pl.ds(0, 16)]
        rd = rowd[b, i, pl.ds(0, 16)]
        ev = (rs[2] + rd[0]) + jnp.zeros((16,), jnp.float32)
        ev = jnp.where(ev >= 0.0, ev, ev * 0.2)
        w = jnp.exp(ev)
        sel = jnp.where(iota == 2, 1.0, rs)
        msg[i, pl.ds(0, 16)] = w * sel
    return edge


def _mid_body(p0_ref, p1_ref, exp8_ref, b1_ref, w2_ref, a2s_ref, a2d_ref,
              haug2_ref, adst2_ref):
    ssum = p0_ref[...] + p1_ref[...]
    num = ssum[:, :F1]
    den = ssum[:, F1:F1 + H1]
    rec = 1.0 / (den + 1e-16)
    rec128 = jnp.dot(rec, exp8_ref[...], preferred_element_type=jnp.float32)
    o1 = num * rec128 + b1_ref[...]
    g = jnp.where(o1 > 0.0, o1, jnp.exp(o1) - 1.0)
    h2 = jnp.dot(g, w2_ref[...], preferred_element_type=jnp.float32)
    s2 = jnp.dot(h2, a2s_ref[...], preferred_element_type=jnp.float32)
    d2 = jnp.dot(h2, a2d_ref[...], preferred_element_type=jnp.float32)
    zb = jnp.zeros((h2.shape[0], 13), jnp.float32)
    zc = jnp.zeros((h2.shape[0], 15), jnp.float32)
    haug2_ref[...] = jnp.concatenate([h2, s2, zb], axis=1)
    adst2_ref[...] = jnp.concatenate([d2, zc], axis=1)


def _mid(part1, EXP8, b1r, W2, a2s, a2d):
    grid = NP // BR
    return pl.pallas_call(
        _mid_body,
        grid=(grid,),
        in_specs=[
            pl.BlockSpec((BR, AUG1), lambda i: (i, 0)),
            pl.BlockSpec((BR, AUG1), lambda i: (i + NP // BR, 0)),
            pl.BlockSpec((H1, F1), lambda i: (0, 0)),
            pl.BlockSpec((1, F1), lambda i: (0, 0)),
            pl.BlockSpec((F1, 2), lambda i: (0, 0)),
            pl.BlockSpec((2, 1), lambda i: (0, 0)),
            pl.BlockSpec((2, 1), lambda i: (0, 0)),
        ],
        out_specs=[
            pl.BlockSpec((BR, AUG2), lambda i: (i, 0)),
            pl.BlockSpec((BR, AUG2), lambda i: (i, 0)),
        ],
        out_shape=[
            jax.ShapeDtypeStruct((NP, AUG2), jnp.float32),
            jax.ShapeDtypeStruct((NP, AUG2), jnp.float32),
        ],
    )(part1, part1, EXP8, b1r, W2, a2s, a2d)


def _fin_body(p0_ref, p1_ref, b2_ref, out_ref):
    ssum = p0_ref[...] + p1_ref[...]
    out_ref[...] = ssum[:, 0:2] / (ssum[:, 2:3] + 1e-16) + b2_ref[...]


def _fin(part2, b2r):
    grid = NP // BR
    return pl.pallas_call(
        _fin_body,
        grid=(grid,),
        in_specs=[
            pl.BlockSpec((BR, AUG2), lambda i: (i, 0)),
            pl.BlockSpec((BR, AUG2), lambda i: (i + NP // BR, 0)),
            pl.BlockSpec((1, 2), lambda i: (0, 0)),
        ],
        out_specs=pl.BlockSpec((BR, 2), lambda i: (i, 0)),
        out_shape=jax.ShapeDtypeStruct((NP, 2), jnp.float32),
    )(part2, part2, b2r)


def kernel(x, edge_index, W1, a1_src, a1_dst, b1, W2, a2_src, a2_dst, b2):
    E = edge_index.shape[1]
    loops = jnp.arange(N, dtype=jnp.int32)
    etot = E + N
    q0, q1 = _split(etot, Q0_FRAC1)
    q2a, q2b = _split(etot, Q0_FRAC2)
    qsum = q0 + q1
    ep = NS * qsum * CH
    junk = jnp.full((ep - etot,), N, jnp.int32)
    srcf = jnp.concatenate([edge_index[0].astype(jnp.int32), loops, junk])
    dstf = jnp.concatenate([edge_index[1].astype(jnp.int32), loops, junk])
    srcc = srcf.reshape(-1, CH)
    dstc = dstf.reshape(-1, CH)

    xp = jnp.pad(x, ((0, NP - N), (0, 0)))
    eye = jnp.eye(H1, dtype=jnp.float32)
    AS = (a1_src[0][:, :, None] * eye[:, None, :]).reshape(F1, H1)
    AD = (a1_dst[0][:, :, None] * eye[:, None, :]).reshape(F1, H1)

    haug1, adst1 = _prep1(xp, W1, AS, AD)
    part1 = _edge_sc(haug1, adst1, srcc, dstc, AUG1, CH, q0, q1, _make_edge1)

    EXP8 = (jnp.arange(F1)[None, :] // C1 == jnp.arange(H1)[:, None]
            ).astype(jnp.float32)
    haug2, adst2 = _mid(part1, EXP8, b1.reshape(1, F1), W2,
                        a2_src.reshape(2, 1), a2_dst.reshape(2, 1))
    part2 = _edge_sc(haug2, adst2, srcc, dstc, AUG2, CH, q2a, q2b, _make_edge2)
    outp = _fin(part2, b2.reshape(1, 2))
    return outp[:N]


# speedup vs baseline: 1.2340x; 1.0030x over previous
import functools

import jax
import jax.numpy as jnp
from jax import lax
from jax.experimental import pallas as pl
from jax.experimental.pallas import tpu as pltpu
from jax.experimental.pallas import tpu_sc as plsc

N = 10000
D_IN = 256
H1, C1 = 8, 16
F1 = H1 * C1
AUG1 = F1 + 16
AUG2 = 16
NP = 10016
NC, NS = 2, 16
NW = NC * NS
CH = 64
BR = 2504
Q0_FRAC1 = 0.762
Q0_FRAC2 = 0.50
RING = 8


def _split(etot, frac):
    qsum = -(-etot // (NS * CH))
    qsum = -(-qsum // RING) * RING
    q0 = int(round(qsum * frac / RING)) * RING
    q0 = min(max(q0, RING), qsum - RING)
    return q0, qsum - q0


def _prep1_body(x_ref, w1_ref, asm_ref, adm_ref, haug_ref, adst_ref):
    h = jnp.dot(x_ref[...], w1_ref[...], preferred_element_type=jnp.float32)
    asrc = jnp.dot(h, asm_ref[...], preferred_element_type=jnp.float32)
    adst = jnp.dot(h, adm_ref[...], preferred_element_type=jnp.float32)
    z8 = jnp.zeros_like(asrc)
    haug_ref[...] = jnp.concatenate([h, asrc, z8], axis=1)
    adst_ref[...] = jnp.concatenate([adst, z8], axis=1)


def _prep1(xp, W1, AS, AD):
    grid = NP // BR
    return pl.pallas_call(
        _prep1_body,
        grid=(grid,),
        in_specs=[
            pl.BlockSpec((BR, D_IN), lambda i: (i, 0)),
            pl.BlockSpec((D_IN, F1), lambda i: (0, 0)),
            pl.BlockSpec((F1, H1), lambda i: (0, 0)),
            pl.BlockSpec((F1, H1), lambda i: (0, 0)),
        ],
        out_specs=[
            pl.BlockSpec((BR, AUG1), lambda i: (i, 0)),
            pl.BlockSpec((BR, 16), lambda i: (i, 0)),
        ],
        out_shape=[
            jax.ShapeDtypeStruct((NP, AUG1), jnp.float32),
            jax.ShapeDtypeStruct((NP, 16), jnp.float32),
        ],
    )(xp, W1, AS, AD)


def _edge_sc(haug, adstt, srcc, dstc, augw, ch, q0, q1, make_edge):
    mesh = plsc.VectorSubcoreMesh(core_axis_name="c", subcore_axis_name="s",
                                  num_cores=NC, num_subcores=NS)
    rpt = NP // NS
    assert q0 % RING == 0 and q1 % RING == 0 and min(q0, q1) >= RING

    @functools.partial(
        pl.kernel,
        out_type=jax.ShapeDtypeStruct((NC * NP, augw), jnp.float32),
        mesh=mesh,
        scratch_types=[
            pltpu.VMEM((RING, ch), jnp.int32),
            pltpu.VMEM((RING, ch), jnp.int32),
            pltpu.VMEM((2, ch, augw), jnp.float32),
            pltpu.VMEM((2, ch, 16), jnp.float32),
            pltpu.VMEM((ch, augw), jnp.float32),
            pltpu.VMEM_SHARED((NP, augw), jnp.float32),
        ] + [pltpu.SemaphoreType.DMA] * (2 * RING + 5),
        compiler_params=pltpu.CompilerParams(use_tc_tiling_on_sc=False),
    )
    def k(haug_hbm, adst_hbm, src_hbm, dst_hbm, out_hbm,
          sib, dib, rows, rowd, msg, acc_sh, *sems_all):
        c = lax.axis_index("c")
        s = lax.axis_index("s")
        semis = list(sems_all[0:RING])
        semid = list(sems_all[RING:2 * RING])
        semg = list(sems_all[2 * RING:2 * RING + 2])
        semd = list(sems_all[2 * RING + 2:2 * RING + 4])
        sems = sems_all[2 * RING + 4]
        qc = jnp.where(c == 0, q0, q1)
        base_chunk = c * (NS * q0) + s * qc

        zero16 = jnp.zeros((16,), jnp.float32)

        def zrow(r, carry):
            for kk in range(augw // 16):
                msg[r, pl.ds(kk * 16, 16)] = zero16
            return carry
        lax.fori_loop(0, ch, zrow, 0)
        off0 = 0
        for sz in ([ch] * (rpt // ch) + ([rpt % ch] if rpt % ch else [])):
            pltpu.sync_copy(msg.at[pl.ds(0, sz)],
                            acc_sh.at[pl.ds(s * rpt + off0, sz)])
            off0 += sz
        plsc.subcore_barrier()

        def fetch_idx(chunk, d):
            pltpu.async_copy(src_hbm.at[base_chunk + chunk], sib.at[d],
                             semis[d])
            pltpu.async_copy(dst_hbm.at[base_chunk + chunk], dib.at[d],
                             semid[d])

        def wait_idx(d):
            pltpu.make_async_copy(src_hbm.at[0], sib.at[d], semis[d]).wait()
            pltpu.make_async_copy(dst_hbm.at[0], dib.at[d], semid[d]).wait()

        def fetch_rows(d, gb):
            pltpu.async_copy(haug_hbm.at[sib.at[d]], rows.at[gb], semg[gb])
            pltpu.async_copy(adst_hbm.at[dib.at[d]], rowd.at[gb], semd[gb])

        def wait_rows(gb):
            pltpu.make_async_copy(haug_hbm.at[pl.ds(0, ch)], rows.at[gb],
                                  semg[gb]).wait()
            pltpu.make_async_copy(adst_hbm.at[pl.ds(0, ch)], rowd.at[gb],
                                  semd[gb]).wait()

        def wait_scatter():
            pltpu.make_async_copy(haug_hbm.at[pl.ds(0, ch)], msg, sems).wait()

        for d in range(RING):
            fetch_idx(d, d)
        for d in range(2):
            wait_idx(d)
            fetch_rows(d, d)

        def handle(g8, b):
            geff = g8 + b
            gb = b & 1
            wait_rows(gb)

            @pl.when(geff >= 1)
            def _():
                wait_scatter()

            @pl.when((geff >= 1) & (geff + (RING - 1) < qc))
            def _():
                fetch_idx(geff + (RING - 1), (b + RING - 1) % RING)

            edge = make_edge(rows, rowd, msg, gb)
            plsc.parallel_loop(0, ch, unroll=4)(edge)

            pltpu.async_copy(msg, acc_sh.at[dib.at[b]], sems, add=True)

            @pl.when(geff + 2 < qc)
            def _():
                wait_idx((b + 2) % RING)
                fetch_rows((b + 2) % RING, gb)

        def octet(p, carry):
            g8 = p * RING
            for b in range(RING):
                handle(g8, b)
            return carry
        lax.fori_loop(0, qc // RING, octet, 0)

        wait_scatter()
        plsc.subcore_barrier()
        off1 = 0
        for sz in ([ch] * (rpt // ch) + ([rpt % ch] if rpt % ch else [])):
            off = s * rpt + off1
            pltpu.sync_copy(acc_sh.at[pl.ds(off, sz)],
                            out_hbm.at[pl.ds(c * NP + off, sz)])
            off1 += sz

    return k(haug, adstt, srcc, dstc)


def _make_edge1(rows, rowd, msg, b):
    iota = lax.iota(jnp.int32, 16)
    mask8 = jnp.where(iota < H1, 1.0, 0.0).astype(jnp.float32)

    def edge(i):
        asrc = rows[b, i, pl.ds(F1, 16)]
        adst = rowd[b, i, pl.ds(0, 16)]
        e = asrc + adst
        e = jnp.where(e >= 0.0, e, e * 0.2)
        w = jnp.exp(e)
        for kk in range(H1):
            msg[i, pl.ds(kk * 16, 16)] = w[kk] * rows[b, i, pl.ds(kk * 16, 16)]
        msg[i, pl.ds(F1, 16)] = w * mask8
    return edge


def _make_edge2(rows, rowd, msg, b):
    iota = lax.iota(jnp.int32, 16)

    def edge(i):
        rs = rows[b, i, pl.ds(0, 16)]
        rd = rowd[b, i, pl.ds(0, 16)]
        ev = (rs[2] + rd[0]) + jnp.zeros((16,), jnp.float32)
        ev = jnp.where(ev >= 0.0, ev, ev * 0.2)
        w = jnp.exp(ev)
        sel = jnp.where(iota == 2, 1.0, rs)
        msg[i, pl.ds(0, 16)] = w * sel
    return edge


def _mid_body(p0_ref, p1_ref, exp8_ref, b1_ref, w2_ref, a2s_ref, a2d_ref,
              haug2_ref, adst2_ref):
    ssum = p0_ref[...] + p1_ref[...]
    num = ssum[:, :F1]
    den = ssum[:, F1:F1 + H1]
    rec = 1.0 / (den + 1e-16)
    rec128 = jnp.dot(rec, exp8_ref[...], preferred_element_type=jnp.float32)
    o1 = num * rec128 + b1_ref[...]
    g = jnp.where(o1 > 0.0, o1, jnp.exp(o1) - 1.0)
    h2 = jnp.dot(g, w2_ref[...], preferred_element_type=jnp.float32)
    s2 = jnp.dot(h2, a2s_ref[...], preferred_element_type=jnp.float32)
    d2 = jnp.dot(h2, a2d_ref[...], preferred_element_type=jnp.float32)
    zb = jnp.zeros((h2.shape[0], 13), jnp.float32)
    zc = jnp.zeros((h2.shape[0], 15), jnp.float32)
    haug2_ref[...] = jnp.concatenate([h2, s2, zb], axis=1)
    adst2_ref[...] = jnp.concatenate([d2, zc], axis=1)


def _mid(part1, EXP8, b1r, W2, a2s, a2d):
    grid = NP // BR
    return pl.pallas_call(
        _mid_body,
        grid=(grid,),
        in_specs=[
            pl.BlockSpec((BR, AUG1), lambda i: (i, 0)),
            pl.BlockSpec((BR, AUG1), lambda i: (i + NP // BR, 0)),
            pl.BlockSpec((H1, F1), lambda i: (0, 0)),
            pl.BlockSpec((1, F1), lambda i: (0, 0)),
            pl.BlockSpec((F1, 2), lambda i: (0, 0)),
            pl.BlockSpec((2, 1), lambda i: (0, 0)),
            pl.BlockSpec((2, 1), lambda i: (0, 0)),
        ],
        out_specs=[
            pl.BlockSpec((BR, AUG2), lambda i: (i, 0)),
            pl.BlockSpec((BR, AUG2), lambda i: (i, 0)),
        ],
        out_shape=[
            jax.ShapeDtypeStruct((NP, AUG2), jnp.float32),
            jax.ShapeDtypeStruct((NP, AUG2), jnp.float32),
        ],
    )(part1, part1, EXP8, b1r, W2, a2s, a2d)


def _fin_body(p0_ref, p1_ref, b2_ref, out_ref):
    ssum = p0_ref[...] + p1_ref[...]
    out_ref[...] = ssum[:, 0:2] / (ssum[:, 2:3] + 1e-16) + b2_ref[...]


def _fin(part2, b2r):
    grid = NP // BR
    return pl.pallas_call(
        _fin_body,
        grid=(grid,),
        in_specs=[
            pl.BlockSpec((BR, AUG2), lambda i: (i, 0)),
            pl.BlockSpec((BR, AUG2), lambda i: (i + NP // BR, 0)),
            pl.BlockSpec((1, 2), lambda i: (0, 0)),
        ],
        out_specs=pl.BlockSpec((BR, 2), lambda i: (i, 0)),
        out_shape=jax.ShapeDtypeStruct((NP, 2), jnp.float32),
    )(part2, part2, b2r)


def kernel(x, edge_index, W1, a1_src, a1_dst, b1, W2, a2_src, a2_dst, b2):
    E = edge_index.shape[1]
    loops = jnp.arange(N, dtype=jnp.int32)
    etot = E + N
    q0, q1 = _split(etot, Q0_FRAC1)
    q2a, q2b = _split(etot, Q0_FRAC2)
    qsum = q0 + q1
    ep = NS * qsum * CH
    junk = jnp.full((ep - etot,), N, jnp.int32)
    srcf = jnp.concatenate([edge_index[0].astype(jnp.int32), loops, junk])
    dstf = jnp.concatenate([edge_index[1].astype(jnp.int32), loops, junk])
    srcc = srcf.reshape(-1, CH)
    dstc = dstf.reshape(-1, CH)

    xp = jnp.pad(x, ((0, NP - N), (0, 0)))
    eye = jnp.eye(H1, dtype=jnp.float32)
    AS = (a1_src[0][:, :, None] * eye[:, None, :]).reshape(F1, H1)
    AD = (a1_dst[0][:, :, None] * eye[:, None, :]).reshape(F1, H1)

    haug1, adst1 = _prep1(xp, W1, AS, AD)
    part1 = _edge_sc(haug1, adst1, srcc, dstc, AUG1, CH, q0, q1, _make_edge1)

    EXP8 = (jnp.arange(F1)[None, :] // C1 == jnp.arange(H1)[:, None]
            ).astype(jnp.float32)
    haug2, adst2 = _mid(part1, EXP8, b1.reshape(1, F1), W2,
                        a2_src.reshape(2, 1), a2_dst.reshape(2, 1))
    part2 = _edge_sc(haug2, adst2, srcc, dstc, AUG2, CH, q2a, q2b, _make_edge2)
    outp = _fin(part2, b2.reshape(1, 2))
    return outp[:N]
